# baseline (speedup 1.0000x reference)
import functools

import jax
import jax.numpy as jnp
from jax import lax
from jax.experimental import pallas as pl
from jax.experimental.pallas import tpu as pltpu
from jax.experimental.pallas import tpu_sc as plsc

MAX_V = 3
D = 128
B = 16384
NC, NS, L = 2, 16, 16
NW = NC * NS
BPW = B // NW
CB = 128
NCH = BPW // CB

_mesh = plsc.VectorSubcoreMesh(
    core_axis_name="c", subcore_axis_name="s", num_cores=NC, num_subcores=NS)


@functools.partial(
    pl.kernel,
    out_type=jax.ShapeDtypeStruct((B, D), jnp.float32),
    mesh=_mesh,
    scratch_types=[
        pltpu.VMEM((MAX_V, D), jnp.float32),
        pltpu.VMEM((L,), jnp.int32),
        pltpu.VMEM((BPW, D), jnp.float32),
    ],
)
def _view_embed_kernel(gf_hbm, emb_hbm, idx_hbm, out_hbm, emb_v, idx_v, buf):
    wid = lax.axis_index("s") * NC + lax.axis_index("c")
    base = wid * BPW

    pltpu.sync_copy(emb_hbm, emb_v)
    pltpu.sync_copy(idx_hbm, idx_v)
    iv = idx_v[...]
    ev = []
    for j in range(D // L):
        r0 = emb_v[0, pl.ds(j * L, L)]
        r1 = emb_v[1, pl.ds(j * L, L)]
        r2 = emb_v[2, pl.ds(j * L, L)]
        ev.append(jnp.where(iv == 0, r0, jnp.where(iv == 1, r1, r2)))

    pltpu.sync_copy(gf_hbm.at[pl.ds(base, BPW)], buf)

    @plsc.parallel_loop(0, BPW, unroll=1)
    def _row(i):
        for j in range(D // L):
            sl = (i, pl.ds(j * L, L))
            buf[sl] = buf[sl] + ev[j]

    pltpu.sync_copy(buf, out_hbm.at[pl.ds(base, BPW)])


def kernel(global_feat, embeddings, view_idx):
    idx = jnp.full((L,), view_idx, dtype=jnp.int32)
    return _view_embed_kernel(global_feat, embeddings, idx)


# speedup vs baseline: 3.0605x; 3.0605x over previous
import functools

import jax
import jax.numpy as jnp
from jax.experimental import pallas as pl
from jax.experimental.pallas import tpu as pltpu

D = 128
B = 16384
BR = 2048


def _body(idx_ref, emb_ref, gf_ref, out_ref):
    idx = idx_ref[0]
    emb_row = emb_ref[pl.ds(idx, 1), :]
    out_ref[...] = gf_ref[...] + emb_row


@jax.jit
def _view_embed(global_feat, embeddings, idx):
    grid = B // BR
    return pl.pallas_call(
        _body,
        grid_spec=pltpu.PrefetchScalarGridSpec(
            num_scalar_prefetch=1,
            grid=(grid,),
            in_specs=[
                pl.BlockSpec((3, D), lambda i, idx: (0, 0)),
                pl.BlockSpec((BR, D), lambda i, idx: (i, 0)),
            ],
            out_specs=pl.BlockSpec((BR, D), lambda i, idx: (i, 0)),
        ),
        out_shape=jax.ShapeDtypeStruct((B, D), jnp.float32),
        compiler_params=pltpu.CompilerParams(
            dimension_semantics=("arbitrary",)),
    )(idx, embeddings, global_feat)


def kernel(global_feat, embeddings, view_idx):
    idx = jnp.asarray(view_idx, dtype=jnp.int32).reshape((1,))
    return _view_embed(global_feat, embeddings, idx)
